# baseline (speedup 1.0000x reference)
import jax
import jax.numpy as jnp
from jax.experimental import pallas as pl
from jax.experimental.pallas import tpu as pltpu

B = 16384
HIDDEN = 256
NUM_SE = 963
BLK = 512
NB = B // BLK


def _body(se_ref, zi_ref, zj_ref, r_ref, d_ref, out_ref):
    idx = se_ref[0, 0, :]
    onehot = (idx[:, None] == jax.lax.broadcasted_iota(
        jnp.int32, (BLK, NUM_SE), 1)).astype(jnp.float32)
    d_r = jax.lax.dot_general(
        onehot, d_ref[...],
        dimension_numbers=(((1,), (0,)), ((), ())),
        preferred_element_type=jnp.float32)
    rz = jax.lax.dot_general(
        zj_ref[...], r_ref[...],
        dimension_numbers=(((1,), (1,)), ((), ())),
        preferred_element_type=jnp.float32)
    prod = zi_ref[...] * rz * (d_r * d_r)
    ones = jnp.ones((HIDDEN, 8), dtype=jnp.float32)
    s = jax.lax.dot_general(
        prod, ones,
        dimension_numbers=(((1,), (0,)), ((), ())),
        preferred_element_type=jnp.float32)
    out_ref[...] = jax.nn.sigmoid(s[:, 0:1])


def kernel(z_i, z_j, R, D, se_indices):
    se3 = se_indices.astype(jnp.int32).reshape(NB, 1, BLK)
    out = pl.pallas_call(
        _body,
        grid=(NB,),
        in_specs=[
            pl.BlockSpec((1, 1, BLK), lambda i: (i, 0, 0)),
            pl.BlockSpec((BLK, HIDDEN), lambda i: (i, 0)),
            pl.BlockSpec((BLK, HIDDEN), lambda i: (i, 0)),
            pl.BlockSpec((HIDDEN, HIDDEN), lambda i: (0, 0)),
            pl.BlockSpec((NUM_SE, HIDDEN), lambda i: (0, 0)),
        ],
        out_specs=pl.BlockSpec((BLK, 1), lambda i: (i, 0)),
        out_shape=jax.ShapeDtypeStruct((B, 1), jnp.float32),
    )(se3, z_i, z_j, R, D)
    return out.reshape(B)


# speedup vs baseline: 1.2295x; 1.2295x over previous
import jax
import jax.numpy as jnp
from jax.experimental import pallas as pl
from jax.experimental.pallas import tpu as pltpu

B = 16384
HIDDEN = 256
NUM_SE = 963
BLK = 512
NB = B // BLK


def _body(se_ref, zi_ref, zj_ref, r_ref, d_ref, out_ref):
    idx = se_ref[0, 0, :]
    onehot = (idx[:, None] == jax.lax.broadcasted_iota(
        jnp.int32, (BLK, NUM_SE), 1)).astype(jnp.float32)
    d_r = jax.lax.dot_general(
        onehot, d_ref[...],
        dimension_numbers=(((1,), (0,)), ((), ())),
        preferred_element_type=jnp.float32)
    rz = jax.lax.dot_general(
        zj_ref[...], r_ref[...],
        dimension_numbers=(((1,), (1,)), ((), ())),
        preferred_element_type=jnp.float32)
    prod = zi_ref[...] * rz * (d_r * d_r)
    ones = jnp.ones((8, HIDDEN), dtype=jnp.float32)
    s = jax.lax.dot_general(
        ones, prod,
        dimension_numbers=(((1,), (1,)), ((), ())),
        preferred_element_type=jnp.float32)
    out_ref[0, 0, :] = jax.nn.sigmoid(s[0, :])


def kernel(z_i, z_j, R, D, se_indices):
    se3 = se_indices.astype(jnp.int32).reshape(NB, 1, BLK)
    out = pl.pallas_call(
        _body,
        grid=(NB,),
        in_specs=[
            pl.BlockSpec((1, 1, BLK), lambda i: (i, 0, 0)),
            pl.BlockSpec((BLK, HIDDEN), lambda i: (i, 0)),
            pl.BlockSpec((BLK, HIDDEN), lambda i: (i, 0)),
            pl.BlockSpec((HIDDEN, HIDDEN), lambda i: (0, 0)),
            pl.BlockSpec((NUM_SE, HIDDEN), lambda i: (0, 0)),
        ],
        out_specs=pl.BlockSpec((1, 1, BLK), lambda i: (i, 0, 0)),
        out_shape=jax.ShapeDtypeStruct((NB, 1, BLK), jnp.float32),
    )(se3, z_i, z_j, R, D)
    return out.reshape(B)


# speedup vs baseline: 1.6526x; 1.3441x over previous
import jax
import jax.numpy as jnp
from jax.experimental import pallas as pl
from jax.experimental.pallas import tpu as pltpu

B = 16384
HIDDEN = 256
NUM_SE = 963
BLK = 1024
NB = B // BLK


def _body(se_ref, zi_ref, zj_ref, r_ref, d_ref, out_ref):
    idx = se_ref[0, 0, :]
    onehot = (idx[:, None] == jax.lax.broadcasted_iota(
        jnp.int32, (BLK, NUM_SE), 1)).astype(jnp.float32)
    d_r = jax.lax.dot_general(
        onehot, d_ref[...],
        dimension_numbers=(((1,), (0,)), ((), ())),
        preferred_element_type=jnp.float32)
    rz = jax.lax.dot_general(
        zj_ref[...], r_ref[...],
        dimension_numbers=(((1,), (1,)), ((), ())),
        preferred_element_type=jnp.float32)
    prod = zi_ref[...] * rz * (d_r * d_r)
    ones = jnp.ones((8, HIDDEN), dtype=jnp.float32)
    s = jax.lax.dot_general(
        ones, prod,
        dimension_numbers=(((1,), (1,)), ((), ())),
        preferred_element_type=jnp.float32)
    out_ref[0, 0, :] = jax.nn.sigmoid(s[0, :])


def kernel(z_i, z_j, R, D, se_indices):
    se3 = se_indices.astype(jnp.int32).reshape(NB, 1, BLK)
    out = pl.pallas_call(
        _body,
        grid=(NB,),
        in_specs=[
            pl.BlockSpec((1, 1, BLK), lambda i: (i, 0, 0)),
            pl.BlockSpec((BLK, HIDDEN), lambda i: (i, 0)),
            pl.BlockSpec((BLK, HIDDEN), lambda i: (i, 0)),
            pl.BlockSpec((HIDDEN, HIDDEN), lambda i: (0, 0)),
            pl.BlockSpec((NUM_SE, HIDDEN), lambda i: (0, 0)),
        ],
        out_specs=pl.BlockSpec((1, 1, BLK), lambda i: (i, 0, 0)),
        out_shape=jax.ShapeDtypeStruct((NB, 1, BLK), jnp.float32),
    )(se3, z_i, z_j, R, D)
    return out.reshape(B)


# speedup vs baseline: 2.0026x; 1.2118x over previous
import jax
import jax.numpy as jnp
from jax.experimental import pallas as pl
from jax.experimental.pallas import tpu as pltpu

B = 16384
HIDDEN = 256
NUM_SE = 963
BLK = 2048
NB = B // BLK


def _body(se_ref, zi_ref, zj_ref, r_ref, d_ref, out_ref):
    idx = se_ref[0, 0, :]
    onehot = (idx[:, None] == jax.lax.broadcasted_iota(
        jnp.int32, (BLK, NUM_SE), 1)).astype(jnp.float32)
    d_r = jax.lax.dot_general(
        onehot, d_ref[...],
        dimension_numbers=(((1,), (0,)), ((), ())),
        preferred_element_type=jnp.float32)
    rz = jax.lax.dot_general(
        zj_ref[...], r_ref[...],
        dimension_numbers=(((1,), (1,)), ((), ())),
        preferred_element_type=jnp.float32)
    prod = zi_ref[...] * rz * (d_r * d_r)
    ones = jnp.ones((8, HIDDEN), dtype=jnp.float32)
    s = jax.lax.dot_general(
        ones, prod,
        dimension_numbers=(((1,), (1,)), ((), ())),
        preferred_element_type=jnp.float32)
    out_ref[0, 0, :] = jax.nn.sigmoid(s[0, :])


def kernel(z_i, z_j, R, D, se_indices):
    se3 = se_indices.astype(jnp.int32).reshape(NB, 1, BLK)
    out = pl.pallas_call(
        _body,
        grid=(NB,),
        in_specs=[
            pl.BlockSpec((1, 1, BLK), lambda i: (i, 0, 0)),
            pl.BlockSpec((BLK, HIDDEN), lambda i: (i, 0)),
            pl.BlockSpec((BLK, HIDDEN), lambda i: (i, 0)),
            pl.BlockSpec((HIDDEN, HIDDEN), lambda i: (0, 0)),
            pl.BlockSpec((NUM_SE, HIDDEN), lambda i: (0, 0)),
        ],
        out_specs=pl.BlockSpec((1, 1, BLK), lambda i: (i, 0, 0)),
        out_shape=jax.ShapeDtypeStruct((NB, 1, BLK), jnp.float32),
    )(se3, z_i, z_j, R, D)
    return out.reshape(B)


# speedup vs baseline: 2.0555x; 1.0264x over previous
import jax
import jax.numpy as jnp
from jax.experimental import pallas as pl
from jax.experimental.pallas import tpu as pltpu

B = 16384
HIDDEN = 256
NUM_SE = 963
BLK = 4096
NB = B // BLK


def _body(se_ref, zi_ref, zj_ref, r_ref, d_ref, out_ref):
    idx = se_ref[0, 0, :]
    onehot = (idx[:, None] == jax.lax.broadcasted_iota(
        jnp.int32, (BLK, NUM_SE), 1)).astype(jnp.float32)
    d_r = jax.lax.dot_general(
        onehot, d_ref[...],
        dimension_numbers=(((1,), (0,)), ((), ())),
        preferred_element_type=jnp.float32)
    rz = jax.lax.dot_general(
        zj_ref[...], r_ref[...],
        dimension_numbers=(((1,), (1,)), ((), ())),
        preferred_element_type=jnp.float32)
    prod = zi_ref[...] * rz * (d_r * d_r)
    ones = jnp.ones((8, HIDDEN), dtype=jnp.float32)
    s = jax.lax.dot_general(
        ones, prod,
        dimension_numbers=(((1,), (1,)), ((), ())),
        preferred_element_type=jnp.float32)
    out_ref[0, 0, :] = jax.nn.sigmoid(s[0, :])


def kernel(z_i, z_j, R, D, se_indices):
    se3 = se_indices.astype(jnp.int32).reshape(NB, 1, BLK)
    out = pl.pallas_call(
        _body,
        grid=(NB,),
        in_specs=[
            pl.BlockSpec((1, 1, BLK), lambda i: (i, 0, 0)),
            pl.BlockSpec((BLK, HIDDEN), lambda i: (i, 0)),
            pl.BlockSpec((BLK, HIDDEN), lambda i: (i, 0)),
            pl.BlockSpec((HIDDEN, HIDDEN), lambda i: (0, 0)),
            pl.BlockSpec((NUM_SE, HIDDEN), lambda i: (0, 0)),
        ],
        out_specs=pl.BlockSpec((1, 1, BLK), lambda i: (i, 0, 0)),
        out_shape=jax.ShapeDtypeStruct((NB, 1, BLK), jnp.float32),
    )(se3, z_i, z_j, R, D)
    return out.reshape(B)
